# Initial kernel scaffold; baseline (speedup 1.0000x reference)
import jax
import jax.numpy as jnp
from jax.experimental import pallas as pl


def kernel(x, edge_index, edge_attr, We, Wc, gamma, beta, Wo, bo):
    raise NotImplementedError("write your pallas kernel here")


# speedup vs baseline: 2.1633x; 2.1633x over previous
import functools

import jax
import jax.numpy as jnp
from jax import lax
from jax.experimental import pallas as pl
from jax.experimental.pallas import tpu as pltpu
from jax.experimental.pallas import tpu_sc as plsc

_N = 10000
_E = 320000
_D = 128
_DE = 16
_L = 4
_NOUT = 128

_NC = 2
_NS = 16
_NW = _NC * _NS
_EPW = _E // _NW
_CB = 80
_NCH = _EPW // _CB
_IBLK = 25
_NIB = _NCH // _IBLK
_NPAD = 10112
_RPS = _NPAD // _NS


def _sc_msgpass(x, ew_l, src3, dst3):
    mesh = plsc.VectorSubcoreMesh(core_axis_name="c", subcore_axis_name="s")

    @functools.partial(
        pl.kernel,
        mesh=mesh,
        out_type=jax.ShapeDtypeStruct((_NC, _NPAD, _D), jnp.float32),
        scratch_types=[
            pltpu.VMEM((_IBLK, _CB), jnp.int32),
            pltpu.VMEM((_IBLK, _CB), jnp.int32),
            pltpu.VMEM((_CB, _D), jnp.float32),
            pltpu.VMEM((_CB, _D), jnp.float32),
            pltpu.VMEM_SHARED((_NPAD, _D), jnp.float32),
            pltpu.SemaphoreType.DMA,
        ],
    )
    def k(x_hbm, ew_hbm, src_hbm, dst_hbm, out_hbm,
          src_v, dst_v, xg_v, ew_v, agg_sh, sem):
        cid = lax.axis_index("c")
        sid = lax.axis_index("s")
        wid = cid * _NS + sid

        def zrow(r, carry):
            z = jnp.zeros((16,), jnp.float32)
            for j in range(_D // 16):
                xg_v[r, pl.ds(j * 16, 16)] = z
            return carry
        lax.fori_loop(0, _CB, zrow, 0)

        base = sid * _RPS
        nfull = _RPS // _CB
        for kk in range(nfull):
            pltpu.sync_copy(xg_v, agg_sh.at[pl.ds(base + kk * _CB, _CB)])
        rem = _RPS - nfull * _CB
        pltpu.sync_copy(xg_v.at[pl.ds(0, rem)],
                        agg_sh.at[pl.ds(base + nfull * _CB, rem)])

        plsc.subcore_barrier()

        ebase = wid * _EPW

        def iblock(g, carry):
            pltpu.sync_copy(src_hbm.at[wid, g], src_v)
            pltpu.sync_copy(dst_hbm.at[wid, g], dst_v)

            def chunk(i, c1):
                pltpu.async_copy(x_hbm.at[src_v.at[i]], xg_v, sem).wait()
                pltpu.sync_copy(
                    ew_hbm.at[pl.ds(ebase + (g * _IBLK + i) * _CB, _CB)],
                    ew_v)

                def crow(r, c2):
                    for j in range(_D // 16):
                        s = pl.ds(j * 16, 16)
                        xg_v[r, s] = jnp.maximum(xg_v[r, s] + ew_v[r, s], 0.0)
                    return c2
                lax.fori_loop(0, _CB, crow, 0)

                pltpu.sync_copy(xg_v, agg_sh.at[dst_v.at[i]], add=True)
                return c1
            lax.fori_loop(0, _IBLK, chunk, 0)
            return carry
        lax.fori_loop(0, _NIB, iblock, 0)

        plsc.subcore_barrier()
        pltpu.sync_copy(agg_sh.at[pl.ds(base, _RPS)],
                        out_hbm.at[cid, pl.ds(base, _RPS)])

    return k(x, ew_l, src3, dst3)


def _tc_edge_embed(edge_attr, We):
    be = 2000
    grid = (_L, _E // be)

    def body(ea_ref, we_ref, o_ref):
        o_ref[...] = jnp.dot(ea_ref[...], we_ref[0],
                             preferred_element_type=jnp.float32)[None]

    return pl.pallas_call(
        body,
        grid=grid,
        in_specs=[
            pl.BlockSpec((be, _DE), lambda l, i: (i, 0)),
            pl.BlockSpec((1, _DE, _D), lambda l, i: (l, 0, 0)),
        ],
        out_specs=pl.BlockSpec((1, be, _D), lambda l, i: (l, i, 0)),
        out_shape=jax.ShapeDtypeStruct((_L, _E, _D), jnp.float32),
    )(edge_attr, We)


def _tc_update(x, aggp, Wc_l, gamma_l, beta_l):
    def body(x_ref, a_ref, w_ref, g_ref, b_ref, o_ref):
        t = x_ref[...] + a_ref[0] + a_ref[1]
        h = jnp.dot(t, w_ref[...], preferred_element_type=jnp.float32)
        mean = jnp.mean(h, axis=0, keepdims=True)
        c = h - mean
        var = jnp.mean(c * c, axis=0, keepdims=True)
        hn = c * lax.rsqrt(var + 1e-5) * g_ref[...] + b_ref[...]
        o_ref[...] = jnp.maximum(hn, 0.0) + x_ref[...]

    return pl.pallas_call(
        body,
        grid=(1,),
        in_specs=[
            pl.BlockSpec((_N, _D), lambda i: (0, 0)),
            pl.BlockSpec((_NC, _N, _D), lambda i: (0, 0, 0)),
            pl.BlockSpec((_D, _D), lambda i: (0, 0)),
            pl.BlockSpec((1, _D), lambda i: (0, 0)),
            pl.BlockSpec((1, _D), lambda i: (0, 0)),
        ],
        out_specs=pl.BlockSpec((_N, _D), lambda i: (0, 0)),
        out_shape=jax.ShapeDtypeStruct((_N, _D), jnp.float32),
    )(x, aggp, Wc_l, gamma_l, beta_l)


def _tc_proj(x, Wo, bo2):
    def body(x_ref, w_ref, b_ref, o_ref):
        o_ref[...] = jnp.dot(x_ref[...], w_ref[...],
                             preferred_element_type=jnp.float32) + b_ref[...]

    return pl.pallas_call(
        body,
        grid=(1,),
        in_specs=[
            pl.BlockSpec((_N, _D), lambda i: (0, 0)),
            pl.BlockSpec((_D, _NOUT), lambda i: (0, 0)),
            pl.BlockSpec((1, _NOUT), lambda i: (0, 0)),
        ],
        out_specs=pl.BlockSpec((_N, _NOUT), lambda i: (0, 0)),
        out_shape=jax.ShapeDtypeStruct((_N, _NOUT), jnp.float32),
    )(x, Wo, bo2)


def kernel(x, edge_index, edge_attr, We, Wc, gamma, beta, Wo, bo):
    src3 = edge_index[0].reshape(_NW, _NIB, _IBLK, _CB)
    dst3 = edge_index[1].reshape(_NW, _NIB, _IBLK, _CB)
    ew = _tc_edge_embed(edge_attr, We)
    for l in range(_L):
        aggp = _sc_msgpass(x, ew[l], src3, dst3)
        x = _tc_update(x, aggp, Wc[l],
                       gamma[l].reshape(1, _D), beta[l].reshape(1, _D))
    return _tc_proj(x, Wo, bo.reshape(1, _NOUT))


# speedup vs baseline: 4.9165x; 2.2727x over previous
import functools

import jax
import jax.numpy as jnp
from jax import lax
from jax.experimental import pallas as pl
from jax.experimental.pallas import tpu as pltpu
from jax.experimental.pallas import tpu_sc as plsc

_N = 10000
_E = 320000
_D = 128
_DE = 16
_L = 4
_NOUT = 128

_NC = 2
_NS = 16
_NW = _NC * _NS
_EPW = _E // _NW
_CB = 40
_NCH = _EPW // _CB
_IBLK = 50
_NIB = _NCH // _IBLK
_NPAD = 10240
_RPS = _NPAD // _NS
_NZC = _RPS // _CB


def _sc_msgpass(x, ew_l, src3, dst3):
    mesh = plsc.VectorSubcoreMesh(core_axis_name="c", subcore_axis_name="s")

    @functools.partial(
        pl.kernel,
        mesh=mesh,
        out_type=jax.ShapeDtypeStruct((_NC, _NPAD, _D), jnp.float32),
        scratch_types=[
            pltpu.VMEM((_IBLK, _CB), jnp.int32),
            pltpu.VMEM((_IBLK, _CB), jnp.int32),
            pltpu.VMEM((2, _CB, _D), jnp.float32),
            pltpu.VMEM((2, _CB, _D), jnp.float32),
            pltpu.VMEM((2, _CB, _D), jnp.float32),
            pltpu.VMEM_SHARED((_NPAD, _D), jnp.float32),
            pltpu.SemaphoreType.DMA,
            pltpu.SemaphoreType.DMA,
            pltpu.SemaphoreType.DMA,
            pltpu.SemaphoreType.DMA,
            pltpu.SemaphoreType.DMA,
            pltpu.SemaphoreType.DMA,
        ],
    )
    def k(x_hbm, ew_hbm, src_hbm, dst_hbm, out_hbm,
          src_v, dst_v, xg_v, ew_v, msg_v,
          agg_sh, sg0, sg1, se0, se1, ss0, ss1):
        cid = lax.axis_index("c")
        sid = lax.axis_index("s")
        wid = cid * _NS + sid
        sg = (sg0, sg1)
        se = (se0, se1)
        ss = (ss0, ss1)

        zb = msg_v.at[0]
        def zrow(r, carry):
            z = jnp.zeros((16,), jnp.float32)
            for j in range(_D // 16):
                zb[r, pl.ds(j * 16, 16)] = z
            return carry
        lax.fori_loop(0, _CB, zrow, 0)

        base = sid * _RPS
        zcopies = [
            pltpu.make_async_copy(
                zb, agg_sh.at[pl.ds(base + kk * _CB, _CB)], ss[kk % 2])
            for kk in range(_NZC)
        ]
        for c in zcopies:
            c.start()
        for c in zcopies:
            c.wait()
        plsc.subcore_barrier()

        ebase = wid * _EPW

        def issue_ge(g, i, b):
            pltpu.make_async_copy(
                x_hbm.at[src_v.at[i]], xg_v.at[b], sg[b]).start()
            pltpu.make_async_copy(
                ew_hbm.at[pl.ds(ebase + (g * _IBLK + i) * _CB, _CB)],
                ew_v.at[b], se[b]).start()

        def wait_ge(b):
            pltpu.make_async_copy(x_hbm.at[src_v.at[0]],
                                  xg_v.at[b], sg[b]).wait()
            pltpu.make_async_copy(ew_hbm.at[pl.ds(0, _CB)],
                                  ew_v.at[b], se[b]).wait()

        def drain_s(b):
            pltpu.make_async_copy(msg_v.at[b],
                                  agg_sh.at[pl.ds(0, _CB)], ss[b]).wait()

        for g in range(_NIB):
            pltpu.sync_copy(src_hbm.at[wid, g], src_v)
            pltpu.sync_copy(dst_hbm.at[wid, g], dst_v)
            issue_ge(g, 0, 0)
            issue_ge(g, 1, 1)

            def pair(p, carry, g=g):
                for b in range(2):
                    i = 2 * p + b
                    wait_ge(b)
                    @pl.when(p > 0)
                    def _(b=b):
                        drain_s(b)

                    def crow(r, c2, b=b):
                        for j in range(_D // 16):
                            s = pl.ds(j * 16, 16)
                            msg_v[b, r, s] = jnp.maximum(
                                xg_v[b, r, s] + ew_v[b, r, s], 0.0)
                        return c2
                    lax.fori_loop(0, _CB, crow, 0)

                    pltpu.async_copy(
                        msg_v.at[b], agg_sh.at[dst_v.at[i]], ss[b],
                        add=True)

                    @pl.when(i + 2 < _IBLK)
                    def _(i=i, b=b, g=g):
                        issue_ge(g, i + 2, b)
                return carry
            lax.fori_loop(0, _IBLK // 2, pair, 0)
            drain_s(0)
            drain_s(1)

        plsc.subcore_barrier()
        pltpu.sync_copy(agg_sh.at[pl.ds(base, _RPS)],
                        out_hbm.at[cid, pl.ds(base, _RPS)])

    return k(x, ew_l, src3, dst3)


def _tc_edge_embed(edge_attr, We_l):
    be = 4000
    grid = (_E // be,)

    def body(ea_ref, we_ref, o_ref):
        o_ref[...] = jnp.dot(ea_ref[...], we_ref[...],
                             preferred_element_type=jnp.float32)

    return pl.pallas_call(
        body,
        grid=grid,
        in_specs=[
            pl.BlockSpec((be, _DE), lambda i: (i, 0)),
            pl.BlockSpec((_DE, _D), lambda i: (0, 0)),
        ],
        out_specs=pl.BlockSpec((be, _D), lambda i: (i, 0)),
        out_shape=jax.ShapeDtypeStruct((_E, _D), jnp.float32),
    )(edge_attr, We_l)


def _tc_update(x, aggp, Wc_l, gamma_l, beta_l):
    def body(x_ref, a_ref, w_ref, g_ref, b_ref, o_ref):
        t = x_ref[...] + a_ref[0] + a_ref[1]
        h = jnp.dot(t, w_ref[...], preferred_element_type=jnp.float32)
        mean = jnp.mean(h, axis=0, keepdims=True)
        c = h - mean
        var = jnp.mean(c * c, axis=0, keepdims=True)
        hn = c * lax.rsqrt(var + 1e-5) * g_ref[...] + b_ref[...]
        o_ref[...] = jnp.maximum(hn, 0.0) + x_ref[...]

    return pl.pallas_call(
        body,
        grid=(1,),
        in_specs=[
            pl.BlockSpec((_N, _D), lambda i: (0, 0)),
            pl.BlockSpec((_NC, _N, _D), lambda i: (0, 0, 0)),
            pl.BlockSpec((_D, _D), lambda i: (0, 0)),
            pl.BlockSpec((1, _D), lambda i: (0, 0)),
            pl.BlockSpec((1, _D), lambda i: (0, 0)),
        ],
        out_specs=pl.BlockSpec((_N, _D), lambda i: (0, 0)),
        out_shape=jax.ShapeDtypeStruct((_N, _D), jnp.float32),
    )(x, aggp, Wc_l, gamma_l, beta_l)


def _tc_proj(x, Wo, bo2):
    def body(x_ref, w_ref, b_ref, o_ref):
        o_ref[...] = jnp.dot(x_ref[...], w_ref[...],
                             preferred_element_type=jnp.float32) + b_ref[...]

    return pl.pallas_call(
        body,
        grid=(1,),
        in_specs=[
            pl.BlockSpec((_N, _D), lambda i: (0, 0)),
            pl.BlockSpec((_D, _NOUT), lambda i: (0, 0)),
            pl.BlockSpec((1, _NOUT), lambda i: (0, 0)),
        ],
        out_specs=pl.BlockSpec((_N, _NOUT), lambda i: (0, 0)),
        out_shape=jax.ShapeDtypeStruct((_N, _NOUT), jnp.float32),
    )(x, Wo, bo2)


def kernel(x, edge_index, edge_attr, We, Wc, gamma, beta, Wo, bo):
    src3 = edge_index[0].reshape(_NW, _NIB, _IBLK, _CB)
    dst3 = edge_index[1].reshape(_NW, _NIB, _IBLK, _CB)
    ew = [_tc_edge_embed(edge_attr, We[l]) for l in range(_L)]
    for l in range(_L):
        aggp = _sc_msgpass(x, ew[l], src3, dst3)
        x = _tc_update(x, aggp, Wc[l],
                       gamma[l].reshape(1, _D), beta[l].reshape(1, _D))
    return _tc_proj(x, Wo, bo.reshape(1, _NOUT))


# speedup vs baseline: 4.9375x; 1.0043x over previous
import functools

import jax
import jax.numpy as jnp
from jax import lax
from jax.experimental import pallas as pl
from jax.experimental.pallas import tpu as pltpu
from jax.experimental.pallas import tpu_sc as plsc

_N = 10000
_E = 320000
_D = 128
_DE = 16
_L = 4
_NOUT = 128

_NC = 2
_NS = 16
_NW = _NC * _NS
_EPW = _E // _NW
_CB = 40
_NCH = _EPW // _CB
_IBLK = 50
_NIB = _NCH // _IBLK
_NPAD = 10240
_RPS = _NPAD // _NS
_NZC = _RPS // _CB


def _sc_msgpass(x, ew_l, src3, dst3):
    mesh = plsc.VectorSubcoreMesh(core_axis_name="c", subcore_axis_name="s")

    @functools.partial(
        pl.kernel,
        mesh=mesh,
        out_type=jax.ShapeDtypeStruct((_NC, _NPAD, _D), jnp.float32),
        scratch_types=[
            pltpu.VMEM((_IBLK, _CB), jnp.int32),
            pltpu.VMEM((_IBLK, _CB), jnp.int32),
            pltpu.VMEM((2, _CB, _D), jnp.float32),
            pltpu.VMEM((2, _CB, _D), jnp.float32),
            pltpu.VMEM((2, _CB, _D), jnp.float32),
            pltpu.VMEM_SHARED((_NPAD, _D), jnp.float32),
            pltpu.SemaphoreType.DMA,
            pltpu.SemaphoreType.DMA,
            pltpu.SemaphoreType.DMA,
            pltpu.SemaphoreType.DMA,
            pltpu.SemaphoreType.DMA,
            pltpu.SemaphoreType.DMA,
        ],
    )
    def k(x_hbm, ew_hbm, src_hbm, dst_hbm, out_hbm,
          src_v, dst_v, xg_v, ew_v, msg_v,
          agg_sh, sg0, sg1, se0, se1, ss0, ss1):
        cid = lax.axis_index("c")
        sid = lax.axis_index("s")
        wid = cid * _NS + sid
        sg = (sg0, sg1)
        se = (se0, se1)
        ss = (ss0, ss1)

        zb = msg_v.at[0]
        def zrow(r, carry):
            z = jnp.zeros((16,), jnp.float32)
            for j in range(_D // 16):
                zb[r, pl.ds(j * 16, 16)] = z
            return carry
        lax.fori_loop(0, _CB, zrow, 0)

        base = sid * _RPS
        zcopies = [
            pltpu.make_async_copy(
                zb, agg_sh.at[pl.ds(base + kk * _CB, _CB)], ss[kk % 2])
            for kk in range(_NZC)
        ]
        for c in zcopies:
            c.start()
        for c in zcopies:
            c.wait()
        plsc.subcore_barrier()

        ebase = wid * _EPW

        def issue_ge(g, i, b):
            pltpu.make_async_copy(
                x_hbm.at[src_v.at[i]], xg_v.at[b], sg[b]).start()
            pltpu.make_async_copy(
                ew_hbm.at[pl.ds(ebase + (g * _IBLK + i) * _CB, _CB)],
                ew_v.at[b], se[b]).start()

        def wait_ge(b):
            pltpu.make_async_copy(x_hbm.at[src_v.at[0]],
                                  xg_v.at[b], sg[b]).wait()
            pltpu.make_async_copy(ew_hbm.at[pl.ds(0, _CB)],
                                  ew_v.at[b], se[b]).wait()

        def drain_s(b):
            pltpu.make_async_copy(msg_v.at[b],
                                  agg_sh.at[pl.ds(0, _CB)], ss[b]).wait()

        for g in range(_NIB):
            pltpu.sync_copy(src_hbm.at[wid, g], src_v)
            pltpu.sync_copy(dst_hbm.at[wid, g], dst_v)
            issue_ge(g, 0, 0)
            issue_ge(g, 1, 1)

            def pair(p, carry, g=g):
                for b in range(2):
                    i = 2 * p + b
                    wait_ge(b)
                    @pl.when(p > 0)
                    def _(b=b):
                        drain_s(b)

                    def crow(r, c2, b=b):
                        for j in range(_D // 16):
                            s = pl.ds(j * 16, 16)
                            msg_v[b, r, s] = jnp.maximum(
                                xg_v[b, r, s] + ew_v[b, r, s], 0.0)
                        return c2
                    lax.fori_loop(0, _CB, crow, 0)

                    pltpu.async_copy(
                        msg_v.at[b], agg_sh.at[dst_v.at[i]], ss[b],
                        add=True)

                    @pl.when(i + 2 < _IBLK)
                    def _(i=i, b=b, g=g):
                        issue_ge(g, i + 2, b)
                return carry
            lax.fori_loop(0, _IBLK // 2, pair, 0)
            drain_s(0)
            drain_s(1)

        plsc.subcore_barrier()
        pltpu.sync_copy(agg_sh.at[pl.ds(base, _RPS)],
                        out_hbm.at[cid, pl.ds(base, _RPS)])

    return k(x, ew_l, src3, dst3)


def _tc_edge_embed(edge_attr, We_l):
    be = 4000
    grid = (_E // be,)

    def body(ea_ref, we_ref, o_ref):
        o_ref[...] = jnp.dot(ea_ref[...], we_ref[...],
                             preferred_element_type=jnp.float32)

    return pl.pallas_call(
        body,
        grid=grid,
        in_specs=[
            pl.BlockSpec((be, _DE), lambda i: (i, 0)),
            pl.BlockSpec((_DE, _D), lambda i: (0, 0)),
        ],
        out_specs=pl.BlockSpec((be, _D), lambda i: (i, 0)),
        out_shape=jax.ShapeDtypeStruct((_E, _D), jnp.float32),
    )(edge_attr, We_l)


def _tc_update(x, aggp, Wc_l, gamma_l, beta_l):
    def body(x_ref, a_ref, w_ref, g_ref, b_ref, o_ref):
        t = x_ref[...] + a_ref[0] + a_ref[1]
        h = jnp.dot(t, w_ref[...], preferred_element_type=jnp.float32)
        mean = jnp.mean(h, axis=0, keepdims=True)
        c = h - mean
        var = jnp.mean(c * c, axis=0, keepdims=True)
        hn = c * lax.rsqrt(var + 1e-5) * g_ref[...] + b_ref[...]
        o_ref[...] = jnp.maximum(hn, 0.0) + x_ref[...]

    return pl.pallas_call(
        body,
        grid=(1,),
        in_specs=[
            pl.BlockSpec((_N, _D), lambda i: (0, 0)),
            pl.BlockSpec((_NC, _N, _D), lambda i: (0, 0, 0)),
            pl.BlockSpec((_D, _D), lambda i: (0, 0)),
            pl.BlockSpec((1, _D), lambda i: (0, 0)),
            pl.BlockSpec((1, _D), lambda i: (0, 0)),
        ],
        out_specs=pl.BlockSpec((_N, _D), lambda i: (0, 0)),
        out_shape=jax.ShapeDtypeStruct((_N, _D), jnp.float32),
    )(x, aggp, Wc_l, gamma_l, beta_l)


def _tc_update_proj(x, aggp, Wc_l, gamma_l, beta_l, Wo, bo2):
    def body(x_ref, a_ref, w_ref, g_ref, b_ref, wo_ref, bo_ref, o_ref):
        t = x_ref[...] + a_ref[0] + a_ref[1]
        h = jnp.dot(t, w_ref[...], preferred_element_type=jnp.float32)
        mean = jnp.mean(h, axis=0, keepdims=True)
        c = h - mean
        var = jnp.mean(c * c, axis=0, keepdims=True)
        hn = c * lax.rsqrt(var + 1e-5) * g_ref[...] + b_ref[...]
        xn = jnp.maximum(hn, 0.0) + x_ref[...]
        o_ref[...] = jnp.dot(
            xn, wo_ref[...], preferred_element_type=jnp.float32) + bo_ref[...]

    return pl.pallas_call(
        body,
        grid=(1,),
        in_specs=[
            pl.BlockSpec((_N, _D), lambda i: (0, 0)),
            pl.BlockSpec((_NC, _N, _D), lambda i: (0, 0, 0)),
            pl.BlockSpec((_D, _D), lambda i: (0, 0)),
            pl.BlockSpec((1, _D), lambda i: (0, 0)),
            pl.BlockSpec((1, _D), lambda i: (0, 0)),
            pl.BlockSpec((_D, _NOUT), lambda i: (0, 0)),
            pl.BlockSpec((1, _NOUT), lambda i: (0, 0)),
        ],
        out_specs=pl.BlockSpec((_N, _NOUT), lambda i: (0, 0)),
        out_shape=jax.ShapeDtypeStruct((_N, _NOUT), jnp.float32),
    )(x, aggp, Wc_l, gamma_l, beta_l, Wo, bo2)


def kernel(x, edge_index, edge_attr, We, Wc, gamma, beta, Wo, bo):
    src3 = edge_index[0].reshape(_NW, _NIB, _IBLK, _CB)
    dst3 = edge_index[1].reshape(_NW, _NIB, _IBLK, _CB)
    ew = [_tc_edge_embed(edge_attr, We[l]) for l in range(_L)]
    for l in range(_L - 1):
        aggp = _sc_msgpass(x, ew[l], src3, dst3)
        x = _tc_update(x, aggp, Wc[l],
                       gamma[l].reshape(1, _D), beta[l].reshape(1, _D))
    aggp = _sc_msgpass(x, ew[_L - 1], src3, dst3)
    return _tc_update_proj(x, aggp, Wc[_L - 1],
                           gamma[_L - 1].reshape(1, _D),
                           beta[_L - 1].reshape(1, _D),
                           Wo, bo.reshape(1, _NOUT))
